# baseline (speedup 1.0000x reference)
import functools

import jax
import jax.numpy as jnp
from jax import lax
from jax.experimental import pallas as pl
from jax.experimental.pallas import tpu as pltpu

try:
    from jax.experimental.pallas import tpu_sc as plsc
    _HAS_SC = True
except ImportError:
    plsc = None
    _HAS_SC = False

N_TOK = 2048
K = 1024
D = 64
M = 4
DP = 128
LOSS_SCALE = 1.25 / (N_TOK * D)

TA = 512
GRID_A2 = N_TOK // TA
TB = 256
GRID = N_TOK // TB


def _topm_body(x_ref, et_ref, cand_ref):
    xb = jnp.concatenate([x_ref[0], x_ref[1]], axis=1)
    et = et_ref[...]

    e2 = jnp.sum(et * et, axis=0, keepdims=True)
    s = lax.dot_general(
        xb, et, (((0,), (0,)), ((), ())),
        precision=lax.Precision.HIGHEST,
        preferred_element_type=jnp.float32)
    d = e2 - (s + s)

    kidx = lax.broadcasted_iota(jnp.int32, (TA, K), 1)
    dpos = jnp.maximum(d * 32.0 + 4.0, 0.5)
    bits = lax.bitcast_convert_type(dpos, jnp.int32)
    keyf = lax.bitcast_convert_type((bits & jnp.int32(~1023)) | kidx,
                                    jnp.float32)
    for j in range(M):
        mn = jnp.min(keyf, axis=1, keepdims=True)
        cand_ref[:, j:j + 1] = (
            lax.bitcast_convert_type(mn, jnp.int32) & 1023)
        if j + 1 < M:
            keyf = jnp.where(keyf == mn, jnp.float32(jnp.inf), keyf)


def _topm(x3, et):
    n = x3.shape[0] * TB
    return pl.pallas_call(
        _topm_body,
        grid=(n // TA,),
        in_specs=[
            pl.BlockSpec((2, D, TB), lambda i: (i, 0, 0)),
            pl.BlockSpec((D, K), lambda i: (0, 0)),
        ],
        out_specs=pl.BlockSpec((TA, M), lambda i: (i, 0)),
        out_shape=jax.ShapeDtypeStruct((n, M), jnp.int32),
    )(x3, et)


def _sc_gather(table_pad, idx, n_rows):
    info = plsc.get_sparse_core_info()
    nw = info.num_cores * info.num_subcores
    b_per_w = n_rows // nw

    mesh = plsc.VectorSubcoreMesh(core_axis_name="c", subcore_axis_name="s")

    @functools.partial(
        pl.kernel,
        mesh=mesh,
        out_type=jax.ShapeDtypeStruct((n_rows, DP), jnp.float32),
        scratch_types=[
            pltpu.VMEM((b_per_w,), jnp.int32),
            pltpu.VMEM((b_per_w, DP), jnp.float32),
            pltpu.SemaphoreType.DMA,
        ],
    )
    def gather_kernel(table_hbm, idx_hbm, out_hbm, idx_v, rows_v, sem):
        wid = lax.axis_index("s") * info.num_cores + lax.axis_index("c")
        base = wid * b_per_w
        pltpu.sync_copy(idx_hbm.at[pl.ds(base, b_per_w)], idx_v)
        pltpu.async_copy(table_hbm.at[idx_v], rows_v, sem).wait()
        pltpu.sync_copy(rows_v, out_hbm.at[pl.ds(base, b_per_w)])

    return gather_kernel(table_pad, idx)


def _winner_body(rows_ref, x_ref, cidx_ref, idx_ref, q_ref, loss_ref):
    i = pl.program_id(0)
    r_raw = rows_ref[...]
    r = jnp.transpose(r_raw[:, :, :D], (0, 2, 1))
    xt = x_ref[0]
    cidx = cidx_ref[...]

    p = r - xt[None, :, :]
    p = p * p
    p4 = p.reshape(M, 8, 8, TB)
    b1 = p4[:, :, 0:4, :] + p4[:, :, 4:8, :]
    b2 = b1[:, :, 0:2, :] + b1[:, :, 2:4, :]
    s = b2[:, :, 0, :] + b2[:, :, 1, :]
    tot = s[:, 0, :]
    for c in range(1, 8):
        tot = tot + s[:, c, :]

    mn = jnp.min(tot, axis=0, keepdims=True)
    big = jnp.int32(2**30)
    widx = jnp.min(jnp.where(tot == mn, cidx, big),
                   axis=0, keepdims=True)
    idx_ref[0, 0, :] = widx[0, :]

    wsel = (tot == mn) & (cidx == widx)
    q_ref[0] = jnp.sum(jnp.where(wsel[:, None, :], r, 0.0), axis=0)

    @pl.when(i == 0)
    def _():
        loss_ref[...] = jnp.zeros_like(loss_ref)

    part = jnp.sum(mn, axis=1, keepdims=True) * LOSS_SCALE
    loss_ref[...] = loss_ref[...] + part


def _winner(rows3, x3, cidx_t):
    grid = x3.shape[0]
    return pl.pallas_call(
        _winner_body,
        grid=(grid,),
        in_specs=[
            pl.BlockSpec((M, TB, DP), lambda i: (0, i, 0)),
            pl.BlockSpec((1, D, TB), lambda i: (i, 0, 0)),
            pl.BlockSpec((M, TB), lambda i: (0, i)),
        ],
        out_specs=[
            pl.BlockSpec((1, 1, TB), lambda i: (i, 0, 0)),
            pl.BlockSpec((1, D, TB), lambda i: (i, 0, 0)),
            pl.BlockSpec((1, 1), lambda i: (0, 0)),
        ],
        out_shape=[
            jax.ShapeDtypeStruct((grid, 1, TB), jnp.int32),
            jax.ShapeDtypeStruct((grid, D, TB), jnp.float32),
            jax.ShapeDtypeStruct((1, 1), jnp.float32),
        ],
    )(rows3, x3, cidx_t)


def kernel(x, embedding_weight):
    x3 = x.reshape(8, 64, 256)
    et = embedding_weight.T
    table_pad = jnp.pad(embedding_weight, ((0, 0), (0, DP - D)))

    half = N_TOK // 2
    outs = []
    for h in range(2):
        x3h = x3[4 * h:4 * h + 4]
        cand = _topm(x3h, et)
        cand_t = cand.T
        flat_idx = cand_t.reshape(half * M)
        rows = _sc_gather(table_pad, flat_idx, half * M)
        rows3 = rows.reshape(M, half, DP)
        outs.append(_winner(rows3, x3h, cand_t))

    (idx_a, q_a, loss_a), (idx_b, q_b, loss_b) = outs
    quantized_out = jnp.concatenate(
        [q_a, q_b], axis=0).reshape(8, 64, 16, 16)
    indices_out = jnp.concatenate(
        [idx_a.reshape(4, 256), idx_b.reshape(4, 256)], axis=0)
    return (loss_a[0, 0] + loss_b[0, 0], quantized_out, indices_out)


# speedup vs baseline: 1.0216x; 1.0216x over previous
import functools

import jax
import jax.numpy as jnp
from jax import lax
from jax.experimental import pallas as pl
from jax.experimental.pallas import tpu as pltpu

try:
    from jax.experimental.pallas import tpu_sc as plsc
    _HAS_SC = True
except ImportError:
    plsc = None
    _HAS_SC = False

N_TOK = 2048
K = 1024
D = 64
M = 4
DP = 128
LOSS_SCALE = 1.25 / (N_TOK * D)

TA = 512
GRID_A2 = N_TOK // TA
TB = 256
GRID = N_TOK // TB


def _topm_body(x_ref, et_ref, cand_ref):
    xb = jnp.concatenate(
        [x_ref[0].reshape(D, TB), x_ref[1].reshape(D, TB)],
        axis=1)
    et = et_ref[...]

    e2 = jnp.sum(et * et, axis=0, keepdims=True)
    xh = xb.astype(jnp.bfloat16)
    xl = (xb - xh.astype(jnp.float32)).astype(jnp.bfloat16)
    eh = et.astype(jnp.bfloat16)
    el = (et - eh.astype(jnp.float32)).astype(jnp.bfloat16)
    dn = (((0,), (0,)), ((), ()))
    s = lax.dot_general(xh, eh, dn, preferred_element_type=jnp.float32)
    s = s + lax.dot_general(xh, el, dn, preferred_element_type=jnp.float32)
    s = s + lax.dot_general(xl, eh, dn, preferred_element_type=jnp.float32)
    d = e2 - (s + s)

    kidx = lax.broadcasted_iota(jnp.int32, (TA, K), 1)
    dpos = jnp.maximum(d * 32.0 + 4.0, 0.5)
    bits = lax.bitcast_convert_type(dpos, jnp.int32)
    keyf = lax.bitcast_convert_type((bits & jnp.int32(~1023)) | kidx,
                                    jnp.float32)
    for j in range(M):
        mn = jnp.min(keyf, axis=1, keepdims=True)
        cand_ref[:, j:j + 1] = (
            lax.bitcast_convert_type(mn, jnp.int32) & 1023)
        if j + 1 < M:
            keyf = jnp.where(keyf == mn, jnp.float32(jnp.inf), keyf)


def _topm(x3, et):
    return pl.pallas_call(
        _topm_body,
        grid=(GRID_A2,),
        in_specs=[
            pl.BlockSpec((2, D, 16, 16), lambda i: (i, 0, 0, 0)),
            pl.BlockSpec((D, K), lambda i: (0, 0)),
        ],
        out_specs=pl.BlockSpec((TA, M), lambda i: (i, 0)),
        out_shape=jax.ShapeDtypeStruct((N_TOK, M), jnp.int32),
    )(x3, et)


def _sc_gather(table_pad, idx, n_rows):
    info = plsc.get_sparse_core_info()
    nw = info.num_cores * info.num_subcores
    b_per_w = n_rows // nw

    mesh = plsc.VectorSubcoreMesh(core_axis_name="c", subcore_axis_name="s")

    @functools.partial(
        pl.kernel,
        mesh=mesh,
        out_type=jax.ShapeDtypeStruct((n_rows, DP), jnp.float32),
        scratch_types=[
            pltpu.VMEM((b_per_w,), jnp.int32),
            pltpu.VMEM((b_per_w, DP), jnp.float32),
            pltpu.SemaphoreType.DMA,
        ],
    )
    def gather_kernel(table_hbm, idx_hbm, out_hbm, idx_v, rows_v, sem):
        wid = lax.axis_index("s") * info.num_cores + lax.axis_index("c")
        base = wid * b_per_w
        pltpu.sync_copy(idx_hbm.at[pl.ds(base, b_per_w)], idx_v)
        pltpu.async_copy(table_hbm.at[idx_v], rows_v, sem).wait()
        pltpu.sync_copy(rows_v, out_hbm.at[pl.ds(base, b_per_w)])

    return gather_kernel(table_pad, idx)


def _winner_body(rows_ref, x_ref, cidx_ref, idx_ref, q_ref, loss_ref):
    i = pl.program_id(0)
    r_raw = rows_ref[...]
    r = jnp.transpose(r_raw[:, :, :D], (0, 2, 1))
    xt = x_ref[0].reshape(D, TB)
    cidx = cidx_ref[...]

    p = r - xt[None, :, :]
    p = p * p
    p4 = p.reshape(M, 8, 8, TB)
    b1 = p4[:, :, 0:4, :] + p4[:, :, 4:8, :]
    b2 = b1[:, :, 0:2, :] + b1[:, :, 2:4, :]
    s = b2[:, :, 0, :] + b2[:, :, 1, :]
    tot = s[:, 0, :]
    for c in range(1, 8):
        tot = tot + s[:, c, :]

    mn = jnp.min(tot, axis=0, keepdims=True)
    big = jnp.int32(2**30)
    widx = jnp.min(jnp.where(tot == mn, cidx, big),
                   axis=0, keepdims=True)
    idx_ref[0, 0, :] = widx[0, :]

    wsel = (tot == mn) & (cidx == widx)
    q_ref[0] = jnp.sum(jnp.where(wsel[:, None, :], r, 0.0), axis=0)

    @pl.when(i == 0)
    def _():
        loss_ref[...] = jnp.zeros_like(loss_ref)

    part = jnp.sum(mn, axis=1, keepdims=True) * LOSS_SCALE
    loss_ref[...] = loss_ref[...] + part


def _winner(rows3, x3, cidx_t):
    return pl.pallas_call(
        _winner_body,
        grid=(GRID,),
        in_specs=[
            pl.BlockSpec((M, TB, DP), lambda i: (0, i, 0)),
            pl.BlockSpec((1, D, 16, 16), lambda i: (i, 0, 0, 0)),
            pl.BlockSpec((M, TB), lambda i: (0, i)),
        ],
        out_specs=[
            pl.BlockSpec((1, 1, TB), lambda i: (i, 0, 0)),
            pl.BlockSpec((1, D, TB), lambda i: (i, 0, 0)),
            pl.BlockSpec((1, 1), lambda i: (0, 0)),
        ],
        out_shape=[
            jax.ShapeDtypeStruct((GRID, 1, TB), jnp.int32),
            jax.ShapeDtypeStruct((GRID, D, TB), jnp.float32),
            jax.ShapeDtypeStruct((1, 1), jnp.float32),
        ],
    )(rows3, x3, cidx_t)


def kernel(x, embedding_weight):
    x3 = x
    et = embedding_weight.T

    cand = _topm(x3, et)
    cand_t = cand.T
    flat_idx = cand_t.reshape(N_TOK * M)

    table_pad = jnp.pad(embedding_weight, ((0, 0), (0, DP - D)))
    rows = _sc_gather(table_pad, flat_idx, N_TOK * M)
    rows3 = rows.reshape(M, N_TOK, DP)

    idx3, q3, loss = _winner(rows3, x3, cand_t)

    quantized_out = q3.reshape(8, 64, 16, 16)
    indices_out = idx3.reshape(8, 256)
    return (loss[0, 0], quantized_out, indices_out)


# speedup vs baseline: 1.1190x; 1.0954x over previous
import functools

import jax
import jax.numpy as jnp
from jax import lax
from jax.experimental import pallas as pl
from jax.experimental.pallas import tpu as pltpu

try:
    from jax.experimental.pallas import tpu_sc as plsc
    _HAS_SC = True
except ImportError:
    plsc = None
    _HAS_SC = False

N_TOK = 2048
K = 1024
D = 64
M = 4
DP = 128
LOSS_SCALE = 1.25 / (N_TOK * D)

TA = 512
GRID_A2 = N_TOK // TA
TB = 256
GRID = N_TOK // TB


def _topm_body(x_ref, et_ref, cand_ref):
    xb = jnp.concatenate([x_ref[0], x_ref[1]], axis=1)
    et = et_ref[...]

    e2 = jnp.sum(et * et, axis=0, keepdims=True)
    xh = xb.astype(jnp.bfloat16)
    xl = (xb - xh.astype(jnp.float32)).astype(jnp.bfloat16)
    eh = et.astype(jnp.bfloat16)
    el = (et - eh.astype(jnp.float32)).astype(jnp.bfloat16)
    dn = (((0,), (0,)), ((), ()))
    s = lax.dot_general(xh, eh, dn, preferred_element_type=jnp.float32)
    s = s + lax.dot_general(xh, el, dn, preferred_element_type=jnp.float32)
    s = s + lax.dot_general(xl, eh, dn, preferred_element_type=jnp.float32)
    d = e2 - (s + s)

    kidx = lax.broadcasted_iota(jnp.int32, (TA, K), 1)
    dpos = jnp.maximum(d * 32.0 + 4.0, 0.5)
    bits = lax.bitcast_convert_type(dpos, jnp.int32)
    keyf = lax.bitcast_convert_type((bits & jnp.int32(~1023)) | kidx,
                                    jnp.float32)
    for j in range(M):
        mn = jnp.min(keyf, axis=1, keepdims=True)
        cand_ref[:, j:j + 1] = (
            lax.bitcast_convert_type(mn, jnp.int32) & 1023)
        if j + 1 < M:
            keyf = jnp.where(keyf == mn, jnp.float32(jnp.inf), keyf)


def _topm(x3, et):
    return pl.pallas_call(
        _topm_body,
        grid=(GRID_A2,),
        in_specs=[
            pl.BlockSpec((2, D, TB), lambda i: (i, 0, 0)),
            pl.BlockSpec((D, K), lambda i: (0, 0)),
        ],
        out_specs=pl.BlockSpec((TA, M), lambda i: (i, 0)),
        out_shape=jax.ShapeDtypeStruct((N_TOK, M), jnp.int32),
    )(x3, et)


def _sc_gather(table_pad, idx, n_rows):
    info = plsc.get_sparse_core_info()
    nw = info.num_cores * info.num_subcores
    b_per_w = n_rows // nw

    mesh = plsc.VectorSubcoreMesh(core_axis_name="c", subcore_axis_name="s")

    @functools.partial(
        pl.kernel,
        mesh=mesh,
        out_type=jax.ShapeDtypeStruct((n_rows, DP), jnp.float32),
        scratch_types=[
            pltpu.VMEM((b_per_w,), jnp.int32),
            pltpu.VMEM((b_per_w, DP), jnp.float32),
            pltpu.SemaphoreType.DMA,
        ],
    )
    def gather_kernel(table_hbm, idx_hbm, out_hbm, idx_v, rows_v, sem):
        wid = lax.axis_index("s") * info.num_cores + lax.axis_index("c")
        base = wid * b_per_w
        pltpu.sync_copy(idx_hbm.at[pl.ds(base, b_per_w)], idx_v)
        pltpu.async_copy(table_hbm.at[idx_v], rows_v, sem).wait()
        pltpu.sync_copy(rows_v, out_hbm.at[pl.ds(base, b_per_w)])

    return gather_kernel(table_pad, idx)


def _winner_body(rows_ref, x_ref, cidx_ref, idx_ref, q_ref, loss_ref):
    i = pl.program_id(0)
    r_raw = rows_ref[...]
    r = jnp.transpose(r_raw[:, :, :D], (0, 2, 1))
    xt = x_ref[0]
    cidx = cidx_ref[...]

    p = r - xt[None, :, :]
    p = p * p
    p4 = p.reshape(M, 8, 8, TB)
    b1 = p4[:, :, 0:4, :] + p4[:, :, 4:8, :]
    b2 = b1[:, :, 0:2, :] + b1[:, :, 2:4, :]
    s = b2[:, :, 0, :] + b2[:, :, 1, :]
    tot = s[:, 0, :]
    for c in range(1, 8):
        tot = tot + s[:, c, :]

    mn = jnp.min(tot, axis=0, keepdims=True)
    big = jnp.int32(2**30)
    widx = jnp.min(jnp.where(tot == mn, cidx, big),
                   axis=0, keepdims=True)
    idx_ref[0, 0, :] = widx[0, :]

    wsel = (tot == mn) & (cidx == widx)
    q_ref[0] = jnp.sum(jnp.where(wsel[:, None, :], r, 0.0), axis=0)

    @pl.when(i == 0)
    def _():
        loss_ref[...] = jnp.zeros_like(loss_ref)

    part = jnp.sum(mn, axis=1, keepdims=True) * LOSS_SCALE
    loss_ref[...] = loss_ref[...] + part


def _winner(rows3, x3, cidx_t):
    return pl.pallas_call(
        _winner_body,
        grid=(GRID,),
        in_specs=[
            pl.BlockSpec((M, TB, DP), lambda i: (0, i, 0)),
            pl.BlockSpec((1, D, TB), lambda i: (i, 0, 0)),
            pl.BlockSpec((M, TB), lambda i: (0, i)),
        ],
        out_specs=[
            pl.BlockSpec((1, 1, TB), lambda i: (i, 0, 0)),
            pl.BlockSpec((1, D, TB), lambda i: (i, 0, 0)),
            pl.BlockSpec((1, 1), lambda i: (0, 0)),
        ],
        out_shape=[
            jax.ShapeDtypeStruct((GRID, 1, TB), jnp.int32),
            jax.ShapeDtypeStruct((GRID, D, TB), jnp.float32),
            jax.ShapeDtypeStruct((1, 1), jnp.float32),
        ],
    )(rows3, x3, cidx_t)


def kernel(x, embedding_weight):
    x3 = x.reshape(8, 64, 256)
    et = embedding_weight.T

    cand = _topm(x3, et)
    cand_t = cand.T
    flat_idx = cand_t.reshape(N_TOK * M)

    table_pad = jnp.pad(embedding_weight, ((0, 0), (0, DP - D)))
    rows = _sc_gather(table_pad, flat_idx, N_TOK * M)
    rows3 = rows.reshape(M, N_TOK, DP)

    idx3, q3, loss = _winner(rows3, x3, cand_t)

    quantized_out = q3.reshape(8, 64, 16, 16)
    indices_out = idx3.reshape(8, 256)
    return (loss[0, 0], quantized_out, indices_out)


# speedup vs baseline: 1.1243x; 1.0047x over previous
import functools

import jax
import jax.numpy as jnp
from jax import lax
from jax.experimental import pallas as pl
from jax.experimental.pallas import tpu as pltpu

try:
    from jax.experimental.pallas import tpu_sc as plsc
    _HAS_SC = True
except ImportError:
    plsc = None
    _HAS_SC = False

N_TOK = 2048
K = 1024
D = 64
M = 4
DP = 128
LOSS_SCALE = 1.25 / (N_TOK * D)

TA = 1024
GRID_A2 = N_TOK // TA
TB = 256
GRID = N_TOK // TB


def _topm_body(x_ref, et_ref, cand_ref):
    xb = jnp.concatenate([x_ref[b] for b in range(TA // TB)],
                         axis=1)
    et = et_ref[...]

    e2 = jnp.sum(et * et, axis=0, keepdims=True)
    xh = xb.astype(jnp.bfloat16)
    xl = (xb - xh.astype(jnp.float32)).astype(jnp.bfloat16)
    eh = et.astype(jnp.bfloat16)
    el = (et - eh.astype(jnp.float32)).astype(jnp.bfloat16)
    dn = (((0,), (0,)), ((), ()))
    s = lax.dot_general(xh, eh, dn, preferred_element_type=jnp.float32)
    s = s + lax.dot_general(xh, el, dn, preferred_element_type=jnp.float32)
    s = s + lax.dot_general(xl, eh, dn, preferred_element_type=jnp.float32)
    d = e2 - (s + s)

    kidx = lax.broadcasted_iota(jnp.int32, (TA, K), 1)
    dpos = jnp.maximum(d * 32.0 + 4.0, 0.5)
    bits = lax.bitcast_convert_type(dpos, jnp.int32)
    keyf = lax.bitcast_convert_type((bits & jnp.int32(~1023)) | kidx,
                                    jnp.float32)
    for j in range(M):
        mn = jnp.min(keyf, axis=1, keepdims=True)
        cand_ref[:, j:j + 1] = (
            lax.bitcast_convert_type(mn, jnp.int32) & 1023)
        if j + 1 < M:
            keyf = jnp.where(keyf == mn, jnp.float32(jnp.inf), keyf)


def _topm(x3, et):
    return pl.pallas_call(
        _topm_body,
        grid=(GRID_A2,),
        in_specs=[
            pl.BlockSpec((TA // TB, D, TB), lambda i: (i, 0, 0)),
            pl.BlockSpec((D, K), lambda i: (0, 0)),
        ],
        out_specs=pl.BlockSpec((TA, M), lambda i: (i, 0)),
        out_shape=jax.ShapeDtypeStruct((N_TOK, M), jnp.int32),
    )(x3, et)


def _sc_gather(table_pad, idx, n_rows):
    info = plsc.get_sparse_core_info()
    nw = info.num_cores * info.num_subcores
    b_per_w = n_rows // nw

    mesh = plsc.VectorSubcoreMesh(core_axis_name="c", subcore_axis_name="s")

    @functools.partial(
        pl.kernel,
        mesh=mesh,
        out_type=jax.ShapeDtypeStruct((n_rows, DP), jnp.float32),
        scratch_types=[
            pltpu.VMEM((b_per_w,), jnp.int32),
            pltpu.VMEM((b_per_w, DP), jnp.float32),
            pltpu.SemaphoreType.DMA,
        ],
    )
    def gather_kernel(table_hbm, idx_hbm, out_hbm, idx_v, rows_v, sem):
        wid = lax.axis_index("s") * info.num_cores + lax.axis_index("c")
        base = wid * b_per_w
        pltpu.sync_copy(idx_hbm.at[pl.ds(base, b_per_w)], idx_v)
        pltpu.async_copy(table_hbm.at[idx_v], rows_v, sem).wait()
        pltpu.sync_copy(rows_v, out_hbm.at[pl.ds(base, b_per_w)])

    return gather_kernel(table_pad, idx)


def _winner_body(rows_ref, x_ref, cidx_ref, idx_ref, q_ref, loss_ref):
    i = pl.program_id(0)
    r_raw = rows_ref[...]
    r = jnp.transpose(r_raw[:, :, :D], (0, 2, 1))
    xt = x_ref[0]
    cidx = cidx_ref[...]

    p = r - xt[None, :, :]
    p = p * p
    p4 = p.reshape(M, 8, 8, TB)
    b1 = p4[:, :, 0:4, :] + p4[:, :, 4:8, :]
    b2 = b1[:, :, 0:2, :] + b1[:, :, 2:4, :]
    s = b2[:, :, 0, :] + b2[:, :, 1, :]
    tot = s[:, 0, :]
    for c in range(1, 8):
        tot = tot + s[:, c, :]

    mn = jnp.min(tot, axis=0, keepdims=True)
    big = jnp.int32(2**30)
    widx = jnp.min(jnp.where(tot == mn, cidx, big),
                   axis=0, keepdims=True)
    idx_ref[0, 0, :] = widx[0, :]

    wsel = (tot == mn) & (cidx == widx)
    q_ref[0] = jnp.sum(jnp.where(wsel[:, None, :], r, 0.0), axis=0)

    @pl.when(i == 0)
    def _():
        loss_ref[...] = jnp.zeros_like(loss_ref)

    part = jnp.sum(mn, axis=1, keepdims=True) * LOSS_SCALE
    loss_ref[...] = loss_ref[...] + part


def _winner(rows3, x3, cidx_t):
    return pl.pallas_call(
        _winner_body,
        grid=(GRID,),
        in_specs=[
            pl.BlockSpec((M, TB, DP), lambda i: (0, i, 0)),
            pl.BlockSpec((1, D, TB), lambda i: (i, 0, 0)),
            pl.BlockSpec((M, TB), lambda i: (0, i)),
        ],
        out_specs=[
            pl.BlockSpec((1, 1, TB), lambda i: (i, 0, 0)),
            pl.BlockSpec((1, D, TB), lambda i: (i, 0, 0)),
            pl.BlockSpec((1, 1), lambda i: (0, 0)),
        ],
        out_shape=[
            jax.ShapeDtypeStruct((GRID, 1, TB), jnp.int32),
            jax.ShapeDtypeStruct((GRID, D, TB), jnp.float32),
            jax.ShapeDtypeStruct((1, 1), jnp.float32),
        ],
    )(rows3, x3, cidx_t)


def kernel(x, embedding_weight):
    x3 = x.reshape(8, 64, 256)
    et = embedding_weight.T

    cand = _topm(x3, et)
    cand_t = cand.T
    flat_idx = cand_t.reshape(N_TOK * M)

    table_pad = jnp.pad(embedding_weight, ((0, 0), (0, DP - D)))
    rows = _sc_gather(table_pad, flat_idx, N_TOK * M)
    rows3 = rows.reshape(M, N_TOK, DP)

    idx3, q3, loss = _winner(rows3, x3, cand_t)

    quantized_out = q3.reshape(8, 64, 16, 16)
    indices_out = idx3.reshape(8, 256)
    return (loss[0, 0], quantized_out, indices_out)
